# baseline (speedup 1.0000x reference)
import jax
import jax.numpy as jnp
from jax import lax
from jax.experimental import pallas as pl
from jax.experimental.pallas import tpu as pltpu
from jax.experimental.pallas import tpu_sc as plsc

N = 10000
E = 320000
D = 128

NC = 2
NS = 16
K = 64
NBUF = 4
CHB = 32
NBLK = 10
CH = CHB * NBLK
E_PAD = NS * K * CH
ROWS_PER_TILE = 640
ACC_ROWS = NS * ROWS_PER_TILE
DUMP = N


def _sc_agg_body(feat_A, feat_B, edges1, edges2,
                 s1_out, d1_out, s2_out, d2_out,
                 acc, deg, idx, r0, r1, r2, r3, ones_v,
                 g0, g1, g2, g3, ss0, ss1, ss2, ss3, dsem):
    rows = [r0, r1, r2, r3]
    gsem = [g0, g1, g2, g3]
    ssem = [ss0, ss1, ss2, ss3]
    cid = lax.axis_index("c")
    sid = lax.axis_index("s")

    zero16 = jnp.zeros((16,), jnp.float32)

    def zrow(i, _):
        for j in range(D // 16):
            r0[i, pl.ds(j * 16, 16)] = zero16
        return 0

    lax.fori_loop(0, K, zrow, 0)
    one16 = jnp.ones((16,), jnp.float32)
    for j in range(K // 16):
        ones_v[pl.ds(j * 16, 16)] = one16

    base = sid * ROWS_PER_TILE
    for k in range(ROWS_PER_TILE // K):
        pltpu.sync_copy(r0, acc.at[pl.ds(base + k * K, K)])
    for k in range(ROWS_PER_TILE // D):
        pltpu.sync_copy(r0.at[0], deg.at[pl.ds(base + k * D, D)])
    plsc.subcore_barrier()

    def run(feat, edg3):
        def block(b, _):
            pltpu.sync_copy(edg3.at[sid, pl.ds(b * CHB, CHB)], idx)

            def lap(jj, _):
                c0 = NBUF * jj
                for s in range(NBUF):
                    @pl.when(jj > 0)
                    def _():
                        pltpu.make_async_copy(
                            rows[s], acc.at[idx.at[c0 + s - NBUF, 1]],
                            ssem[s]).wait()
                    pltpu.async_copy(feat.at[idx.at[c0 + s, 0]], rows[s],
                                     gsem[s])
                for s in range(NBUF):
                    pltpu.make_async_copy(feat.at[idx.at[c0 + s, 0]],
                                          rows[s], gsem[s]).wait()
                    pltpu.async_copy(rows[s], acc.at[idx.at[c0 + s, 1]],
                                     ssem[s], add=True)
                    pltpu.async_copy(ones_v, deg.at[idx.at[c0 + s, 1]],
                                     dsem, add=True)

                @pl.when(jj > 0)
                def _():
                    for s in range(NBUF):
                        pltpu.make_async_copy(
                            ones_v, deg.at[idx.at[c0 + s - NBUF, 1]],
                            dsem).wait()
                return 0

            lax.fori_loop(0, CHB // NBUF, lap, 0)
            for s in range(NBUF):
                pltpu.make_async_copy(
                    rows[s], acc.at[idx.at[CHB - NBUF + s, 1]],
                    ssem[s]).wait()
                pltpu.make_async_copy(
                    ones_v, deg.at[idx.at[CHB - NBUF + s, 1]], dsem).wait()
            return 0

        lax.fori_loop(0, NBLK, block, 0)

    @pl.when(cid == 0)
    def _():
        run(feat_A, edges1)

    @pl.when(cid == 1)
    def _():
        run(feat_B, edges2)

    plsc.subcore_barrier()

    @pl.when(cid == 0)
    def _():
        pltpu.sync_copy(acc.at[pl.ds(base, ROWS_PER_TILE)],
                        s1_out.at[pl.ds(base, ROWS_PER_TILE)])
        pltpu.sync_copy(deg.at[pl.ds(base, ROWS_PER_TILE)],
                        d1_out.at[pl.ds(base, ROWS_PER_TILE)])

    @pl.when(cid == 1)
    def _():
        pltpu.sync_copy(acc.at[pl.ds(base, ROWS_PER_TILE)],
                        s2_out.at[pl.ds(base, ROWS_PER_TILE)])
        pltpu.sync_copy(deg.at[pl.ds(base, ROWS_PER_TILE)],
                        d2_out.at[pl.ds(base, ROWS_PER_TILE)])


def _sc_aggregate(feat_A, feat_B, edges1, edges2):
    mesh = plsc.VectorSubcoreMesh(core_axis_name="c", subcore_axis_name="s",
                                  num_cores=NC, num_subcores=NS)
    f32 = jnp.float32
    out_type = (
        jax.ShapeDtypeStruct((ACC_ROWS, D), f32),
        jax.ShapeDtypeStruct((ACC_ROWS,), f32),
        jax.ShapeDtypeStruct((ACC_ROWS, D), f32),
        jax.ShapeDtypeStruct((ACC_ROWS,), f32),
    )
    scratch = [
        pltpu.VMEM_SHARED((ACC_ROWS, D), f32),
        pltpu.VMEM_SHARED((ACC_ROWS,), f32),
        pltpu.VMEM((CHB, 2, K), jnp.int32),
        pltpu.VMEM((K, D), f32),
        pltpu.VMEM((K, D), f32),
        pltpu.VMEM((K, D), f32),
        pltpu.VMEM((K, D), f32),
        pltpu.VMEM((K,), f32),
        pltpu.SemaphoreType.DMA,
        pltpu.SemaphoreType.DMA,
        pltpu.SemaphoreType.DMA,
        pltpu.SemaphoreType.DMA,
        pltpu.SemaphoreType.DMA,
        pltpu.SemaphoreType.DMA,
        pltpu.SemaphoreType.DMA,
        pltpu.SemaphoreType.DMA,
        pltpu.SemaphoreType.DMA,
    ]
    fn = pl.kernel(_sc_agg_body, out_type=out_type, mesh=mesh,
                   scratch_types=scratch)
    return fn(feat_A, feat_B, edges1, edges2)


def _combine_body(s1_ref, d1_ref, s2_ref, d2_ref, w1_ref, b1_ref,
                  w2_ref, b2_ref, out_ref):
    d1 = d1_ref[...]
    d2 = d2_ref[...]
    x1 = s1_ref[...] / jnp.maximum(d1, 1.0)
    x2 = s2_ref[...] / jnp.maximum(d2, 1.0)
    h = jnp.dot(x1, w1_ref[...], preferred_element_type=jnp.float32)
    h += jnp.dot(x2, w2_ref[...], preferred_element_type=jnp.float32)
    h += jnp.where(d1 > 0, b1_ref[...], 0.0)
    h += jnp.where(d2 > 0, b2_ref[...], 0.0)
    out_ref[...] = h


def _combine(s1, deg1, s2, deg2, W_e1, b_e1, W_e2, b_e2):
    BLK = 400
    grid = (N // BLK,)
    d1 = deg1.reshape(ACC_ROWS, 1)
    d2 = deg2.reshape(ACC_ROWS, 1)
    b1 = b_e1.reshape(1, D)
    b2 = b_e2.reshape(1, D)
    row_spec = pl.BlockSpec((BLK, D), lambda i: (i, 0))
    deg_spec = pl.BlockSpec((BLK, 1), lambda i: (i, 0))
    full_w = pl.BlockSpec((D, D), lambda i: (0, 0))
    full_b = pl.BlockSpec((1, D), lambda i: (0, 0))
    return pl.pallas_call(
        _combine_body,
        grid=grid,
        in_specs=[row_spec, deg_spec, row_spec, deg_spec,
                  full_w, full_b, full_w, full_b],
        out_specs=pl.BlockSpec((BLK, D), lambda i: (i, 0)),
        out_shape=jax.ShapeDtypeStruct((N, D), jnp.float32),
    )(s1, d1, s2, d2, W_e1, b1, W_e2, b2)


def _pad_edges(edge):
    pad = E_PAD - E
    src = jnp.concatenate([edge[0], jnp.zeros((pad,), jnp.int32)])
    dst = jnp.concatenate([edge[1], jnp.full((pad,), DUMP, jnp.int32)])
    return jnp.stack([src.reshape(NS, CH, K), dst.reshape(NS, CH, K)],
                     axis=2)


@jax.jit
def kernel(feat_A, feat_B, edge_e1, edge_e2, W_e1, b_e1, W_e2, b_e2):
    edges1 = _pad_edges(edge_e1)
    edges2 = _pad_edges(edge_e2)
    s1, d1, s2, d2 = _sc_aggregate(feat_A, feat_B, edges1, edges2)
    return _combine(s1, d1, s2, d2, W_e1, b_e1, W_e2, b_e2)


# speedup vs baseline: 1.1373x; 1.1373x over previous
import jax
import jax.numpy as jnp
from jax import lax
from jax.experimental import pallas as pl
from jax.experimental.pallas import tpu as pltpu
from jax.experimental.pallas import tpu_sc as plsc

N = 10000
E = 320000
D = 128

NC = 2
NS = 16
K = 128
CHB = 40
NBLK = 4
CH = CHB * NBLK
E_PAD = NS * K * CH
ROWS_PER_TILE = 640
ACC_ROWS = NS * ROWS_PER_TILE
DUMP = N


def _sc_agg_body(feat_A, feat_B, src1, dst1, src2, dst2,
                 s1_out, d1_out, s2_out, d2_out,
                 acc, deg, isrc, idst, rows0, rows1, ones_v,
                 gsem0, gsem1, ssem0, ssem1, dsem):
    cid = lax.axis_index("c")
    sid = lax.axis_index("s")

    zero16 = jnp.zeros((16,), jnp.float32)

    def zrow(i, _):
        for j in range(D // 16):
            rows0[i, pl.ds(j * 16, 16)] = zero16
        return 0

    lax.fori_loop(0, K, zrow, 0)
    one16 = jnp.ones((16,), jnp.float32)
    for j in range(K // 16):
        ones_v[pl.ds(j * 16, 16)] = one16

    base = sid * ROWS_PER_TILE
    for k in range(ROWS_PER_TILE // K):
        pltpu.sync_copy(rows0, acc.at[pl.ds(base + k * K, K)])
        pltpu.sync_copy(rows0.at[0], deg.at[pl.ds(base + k * K, K)])
    plsc.subcore_barrier()

    def run(feat, esrc, edst):
        def block(b, _):
            pltpu.sync_copy(esrc.at[sid, pl.ds(b * CHB, CHB)], isrc)
            pltpu.sync_copy(edst.at[sid, pl.ds(b * CHB, CHB)], idst)

            def pair(jj, _):
                a = 2 * jj

                @pl.when(jj > 0)
                def _():
                    pltpu.make_async_copy(
                        rows0, acc.at[idst.at[a - 2]], ssem0).wait()
                pltpu.async_copy(feat.at[isrc.at[a]], rows0, gsem0)

                @pl.when(jj > 0)
                def _():
                    pltpu.make_async_copy(
                        rows1, acc.at[idst.at[a - 1]], ssem1).wait()
                pltpu.async_copy(feat.at[isrc.at[a + 1]], rows1, gsem1)

                pltpu.make_async_copy(feat.at[isrc.at[a]], rows0,
                                      gsem0).wait()
                pltpu.async_copy(rows0, acc.at[idst.at[a]], ssem0,
                                 add=True)
                pltpu.async_copy(ones_v, deg.at[idst.at[a]], dsem,
                                 add=True)

                pltpu.make_async_copy(feat.at[isrc.at[a + 1]], rows1,
                                      gsem1).wait()
                pltpu.async_copy(rows1, acc.at[idst.at[a + 1]], ssem1,
                                 add=True)
                pltpu.async_copy(ones_v, deg.at[idst.at[a + 1]], dsem,
                                 add=True)

                @pl.when(jj > 0)
                def _():
                    pltpu.make_async_copy(
                        ones_v, deg.at[idst.at[a - 2]], dsem).wait()
                    pltpu.make_async_copy(
                        ones_v, deg.at[idst.at[a - 1]], dsem).wait()
                return 0

            lax.fori_loop(0, CHB // 2, pair, 0)
            pltpu.make_async_copy(rows0, acc.at[idst.at[CHB - 2]],
                                  ssem0).wait()
            pltpu.make_async_copy(rows1, acc.at[idst.at[CHB - 1]],
                                  ssem1).wait()
            pltpu.make_async_copy(ones_v, deg.at[idst.at[CHB - 2]],
                                  dsem).wait()
            pltpu.make_async_copy(ones_v, deg.at[idst.at[CHB - 1]],
                                  dsem).wait()
            return 0

        lax.fori_loop(0, NBLK, block, 0)

    @pl.when(cid == 0)
    def _():
        run(feat_A, src1, dst1)

    @pl.when(cid == 1)
    def _():
        run(feat_B, src2, dst2)

    plsc.subcore_barrier()

    @pl.when(cid == 0)
    def _():
        pltpu.sync_copy(acc.at[pl.ds(base, ROWS_PER_TILE)],
                        s1_out.at[pl.ds(base, ROWS_PER_TILE)])
        pltpu.sync_copy(deg.at[pl.ds(base, ROWS_PER_TILE)],
                        d1_out.at[pl.ds(base, ROWS_PER_TILE)])

    @pl.when(cid == 1)
    def _():
        pltpu.sync_copy(acc.at[pl.ds(base, ROWS_PER_TILE)],
                        s2_out.at[pl.ds(base, ROWS_PER_TILE)])
        pltpu.sync_copy(deg.at[pl.ds(base, ROWS_PER_TILE)],
                        d2_out.at[pl.ds(base, ROWS_PER_TILE)])


def _sc_aggregate(feat_A, feat_B, src1, dst1, src2, dst2):
    mesh = plsc.VectorSubcoreMesh(core_axis_name="c", subcore_axis_name="s",
                                  num_cores=NC, num_subcores=NS)
    f32 = jnp.float32
    out_type = (
        jax.ShapeDtypeStruct((ACC_ROWS, D), f32),
        jax.ShapeDtypeStruct((ACC_ROWS,), f32),
        jax.ShapeDtypeStruct((ACC_ROWS, D), f32),
        jax.ShapeDtypeStruct((ACC_ROWS,), f32),
    )
    scratch = [
        pltpu.VMEM_SHARED((ACC_ROWS, D), f32),
        pltpu.VMEM_SHARED((ACC_ROWS,), f32),
        pltpu.VMEM((CHB, K), jnp.int32),
        pltpu.VMEM((CHB, K), jnp.int32),
        pltpu.VMEM((K, D), f32),
        pltpu.VMEM((K, D), f32),
        pltpu.VMEM((K,), f32),
        pltpu.SemaphoreType.DMA,
        pltpu.SemaphoreType.DMA,
        pltpu.SemaphoreType.DMA,
        pltpu.SemaphoreType.DMA,
        pltpu.SemaphoreType.DMA,
    ]
    fn = pl.kernel(_sc_agg_body, out_type=out_type, mesh=mesh,
                   scratch_types=scratch)
    return fn(feat_A, feat_B, src1, dst1, src2, dst2)


def _combine_body(s1_ref, d1_ref, s2_ref, d2_ref, w1_ref, b1_ref,
                  w2_ref, b2_ref, out_ref):
    d1 = d1_ref[...]
    d2 = d2_ref[...]
    x1 = s1_ref[...] / jnp.maximum(d1, 1.0)
    x2 = s2_ref[...] / jnp.maximum(d2, 1.0)
    h = jnp.dot(x1, w1_ref[...], preferred_element_type=jnp.float32)
    h += jnp.dot(x2, w2_ref[...], preferred_element_type=jnp.float32)
    h += jnp.where(d1 > 0, b1_ref[...], 0.0)
    h += jnp.where(d2 > 0, b2_ref[...], 0.0)
    out_ref[...] = h


def _combine(s1, deg1, s2, deg2, W_e1, b_e1, W_e2, b_e2):
    BLK = 400
    grid = (N // BLK,)
    d1 = deg1.reshape(ACC_ROWS, 1)
    d2 = deg2.reshape(ACC_ROWS, 1)
    b1 = b_e1.reshape(1, D)
    b2 = b_e2.reshape(1, D)
    row_spec = pl.BlockSpec((BLK, D), lambda i: (i, 0))
    deg_spec = pl.BlockSpec((BLK, 1), lambda i: (i, 0))
    full_w = pl.BlockSpec((D, D), lambda i: (0, 0))
    full_b = pl.BlockSpec((1, D), lambda i: (0, 0))
    return pl.pallas_call(
        _combine_body,
        grid=grid,
        in_specs=[row_spec, deg_spec, row_spec, deg_spec,
                  full_w, full_b, full_w, full_b],
        out_specs=pl.BlockSpec((BLK, D), lambda i: (i, 0)),
        out_shape=jax.ShapeDtypeStruct((N, D), jnp.float32),
    )(s1, d1, s2, d2, W_e1, b1, W_e2, b2)


def _pad_edges(edge):
    pad = E_PAD - E
    src = jnp.concatenate([edge[0], jnp.zeros((pad,), jnp.int32)])
    dst = jnp.concatenate([edge[1], jnp.full((pad,), DUMP, jnp.int32)])
    return src.reshape(NS, CH, K), dst.reshape(NS, CH, K)


@jax.jit
def kernel(feat_A, feat_B, edge_e1, edge_e2, W_e1, b_e1, W_e2, b_e2):
    src1, dst1 = _pad_edges(edge_e1)
    src2, dst2 = _pad_edges(edge_e2)
    s1, d1, s2, d2 = _sc_aggregate(feat_A, feat_B, src1, dst1, src2, dst2)
    return _combine(s1, d1, s2, d2, W_e1, b_e1, W_e2, b_e2)


# speedup vs baseline: 1.1389x; 1.0014x over previous
import jax
import jax.numpy as jnp
from jax import lax
from jax.experimental import pallas as pl
from jax.experimental.pallas import tpu as pltpu
from jax.experimental.pallas import tpu_sc as plsc

N = 10000
E = 320000
D = 128

NC = 2
NS = 16
K = 128
CHB = 40
NBLK = 4
CH = CHB * NBLK
E_PAD = NS * K * CH
ROWS_PER_TILE = 640
ACC_ROWS = NS * ROWS_PER_TILE
DUMP = N


def _sc_agg_body(feat_A, feat_B, src1, dst1, src2, dst2,
                 s1_out, d1_out, s2_out, d2_out,
                 acc, deg, isrc, idst, rows0, rows1, ones_v,
                 gsem0, gsem1, ssem0, ssem1, dsem):
    cid = lax.axis_index("c")
    sid = lax.axis_index("s")

    zero16 = jnp.zeros((16,), jnp.float32)

    def zrow(i, _):
        for j in range(D // 16):
            rows0[i, pl.ds(j * 16, 16)] = zero16
        return 0

    lax.fori_loop(0, K, zrow, 0)
    one16 = jnp.ones((16,), jnp.float32)
    for j in range(K // 16):
        ones_v[pl.ds(j * 16, 16)] = one16

    base = sid * ROWS_PER_TILE
    for k in range(ROWS_PER_TILE // K):
        pltpu.async_copy(rows0, acc.at[pl.ds(base + k * K, K)], gsem0)
        pltpu.async_copy(rows0.at[0], deg.at[pl.ds(base + k * K, K)], gsem1)
    for k in range(ROWS_PER_TILE // K):
        pltpu.make_async_copy(rows0, acc.at[pl.ds(base + k * K, K)],
                              gsem0).wait()
        pltpu.make_async_copy(rows0.at[0], deg.at[pl.ds(base + k * K, K)],
                              gsem1).wait()
    plsc.subcore_barrier()

    def run(feat, esrc, edst):
        def block(b, _):
            pltpu.sync_copy(esrc.at[sid, pl.ds(b * CHB, CHB)], isrc)
            pltpu.sync_copy(edst.at[sid, pl.ds(b * CHB, CHB)], idst)

            def pair(jj, _):
                a = 2 * jj

                @pl.when(jj > 0)
                def _():
                    pltpu.make_async_copy(
                        rows0, acc.at[idst.at[a - 2]], ssem0).wait()
                pltpu.async_copy(feat.at[isrc.at[a]], rows0, gsem0)

                @pl.when(jj > 0)
                def _():
                    pltpu.make_async_copy(
                        rows1, acc.at[idst.at[a - 1]], ssem1).wait()
                pltpu.async_copy(feat.at[isrc.at[a + 1]], rows1, gsem1)

                pltpu.make_async_copy(feat.at[isrc.at[a]], rows0,
                                      gsem0).wait()
                pltpu.async_copy(rows0, acc.at[idst.at[a]], ssem0,
                                 add=True)
                pltpu.async_copy(ones_v, deg.at[idst.at[a]], dsem,
                                 add=True)

                pltpu.make_async_copy(feat.at[isrc.at[a + 1]], rows1,
                                      gsem1).wait()
                pltpu.async_copy(rows1, acc.at[idst.at[a + 1]], ssem1,
                                 add=True)
                pltpu.async_copy(ones_v, deg.at[idst.at[a + 1]], dsem,
                                 add=True)

                @pl.when(jj > 0)
                def _():
                    pltpu.make_async_copy(
                        ones_v, deg.at[idst.at[a - 2]], dsem).wait()
                    pltpu.make_async_copy(
                        ones_v, deg.at[idst.at[a - 1]], dsem).wait()
                return 0

            lax.fori_loop(0, CHB // 2, pair, 0)
            pltpu.make_async_copy(rows0, acc.at[idst.at[CHB - 2]],
                                  ssem0).wait()
            pltpu.make_async_copy(rows1, acc.at[idst.at[CHB - 1]],
                                  ssem1).wait()
            pltpu.make_async_copy(ones_v, deg.at[idst.at[CHB - 2]],
                                  dsem).wait()
            pltpu.make_async_copy(ones_v, deg.at[idst.at[CHB - 1]],
                                  dsem).wait()
            return 0

        lax.fori_loop(0, NBLK, block, 0)

    @pl.when(cid == 0)
    def _():
        run(feat_A, src1, dst1)

    @pl.when(cid == 1)
    def _():
        run(feat_B, src2, dst2)

    plsc.subcore_barrier()

    @pl.when(cid == 0)
    def _():
        pltpu.async_copy(acc.at[pl.ds(base, ROWS_PER_TILE)],
                         s1_out.at[pl.ds(base, ROWS_PER_TILE)], gsem0)
        pltpu.async_copy(deg.at[pl.ds(base, ROWS_PER_TILE)],
                         d1_out.at[pl.ds(base, ROWS_PER_TILE)], gsem1)
        pltpu.make_async_copy(acc.at[pl.ds(base, ROWS_PER_TILE)],
                              s1_out.at[pl.ds(base, ROWS_PER_TILE)],
                              gsem0).wait()
        pltpu.make_async_copy(deg.at[pl.ds(base, ROWS_PER_TILE)],
                              d1_out.at[pl.ds(base, ROWS_PER_TILE)],
                              gsem1).wait()

    @pl.when(cid == 1)
    def _():
        pltpu.async_copy(acc.at[pl.ds(base, ROWS_PER_TILE)],
                         s2_out.at[pl.ds(base, ROWS_PER_TILE)], gsem0)
        pltpu.async_copy(deg.at[pl.ds(base, ROWS_PER_TILE)],
                         d2_out.at[pl.ds(base, ROWS_PER_TILE)], gsem1)
        pltpu.make_async_copy(acc.at[pl.ds(base, ROWS_PER_TILE)],
                              s2_out.at[pl.ds(base, ROWS_PER_TILE)],
                              gsem0).wait()
        pltpu.make_async_copy(deg.at[pl.ds(base, ROWS_PER_TILE)],
                              d2_out.at[pl.ds(base, ROWS_PER_TILE)],
                              gsem1).wait()


def _sc_aggregate(feat_A, feat_B, src1, dst1, src2, dst2):
    mesh = plsc.VectorSubcoreMesh(core_axis_name="c", subcore_axis_name="s",
                                  num_cores=NC, num_subcores=NS)
    f32 = jnp.float32
    out_type = (
        jax.ShapeDtypeStruct((ACC_ROWS, D), f32),
        jax.ShapeDtypeStruct((ACC_ROWS,), f32),
        jax.ShapeDtypeStruct((ACC_ROWS, D), f32),
        jax.ShapeDtypeStruct((ACC_ROWS,), f32),
    )
    scratch = [
        pltpu.VMEM_SHARED((ACC_ROWS, D), f32),
        pltpu.VMEM_SHARED((ACC_ROWS,), f32),
        pltpu.VMEM((CHB, K), jnp.int32),
        pltpu.VMEM((CHB, K), jnp.int32),
        pltpu.VMEM((K, D), f32),
        pltpu.VMEM((K, D), f32),
        pltpu.VMEM((K,), f32),
        pltpu.SemaphoreType.DMA,
        pltpu.SemaphoreType.DMA,
        pltpu.SemaphoreType.DMA,
        pltpu.SemaphoreType.DMA,
        pltpu.SemaphoreType.DMA,
    ]
    fn = pl.kernel(_sc_agg_body, out_type=out_type, mesh=mesh,
                   scratch_types=scratch)
    return fn(feat_A, feat_B, src1, dst1, src2, dst2)


def _combine_body(s1_ref, d1_ref, s2_ref, d2_ref, w1_ref, b1_ref,
                  w2_ref, b2_ref, out_ref):
    d1 = d1_ref[...]
    d2 = d2_ref[...]
    x1 = s1_ref[...] / jnp.maximum(d1, 1.0)
    x2 = s2_ref[...] / jnp.maximum(d2, 1.0)
    h = jnp.dot(x1, w1_ref[...], preferred_element_type=jnp.float32)
    h += jnp.dot(x2, w2_ref[...], preferred_element_type=jnp.float32)
    h += jnp.where(d1 > 0, b1_ref[...], 0.0)
    h += jnp.where(d2 > 0, b2_ref[...], 0.0)
    out_ref[...] = h


def _combine(s1, deg1, s2, deg2, W_e1, b_e1, W_e2, b_e2):
    BLK = 400
    grid = (N // BLK,)
    d1 = deg1.reshape(ACC_ROWS, 1)
    d2 = deg2.reshape(ACC_ROWS, 1)
    b1 = b_e1.reshape(1, D)
    b2 = b_e2.reshape(1, D)
    row_spec = pl.BlockSpec((BLK, D), lambda i: (i, 0))
    deg_spec = pl.BlockSpec((BLK, 1), lambda i: (i, 0))
    full_w = pl.BlockSpec((D, D), lambda i: (0, 0))
    full_b = pl.BlockSpec((1, D), lambda i: (0, 0))
    return pl.pallas_call(
        _combine_body,
        grid=grid,
        in_specs=[row_spec, deg_spec, row_spec, deg_spec,
                  full_w, full_b, full_w, full_b],
        out_specs=pl.BlockSpec((BLK, D), lambda i: (i, 0)),
        out_shape=jax.ShapeDtypeStruct((N, D), jnp.float32),
    )(s1, d1, s2, d2, W_e1, b1, W_e2, b2)


def _pad_edges(edge):
    pad = E_PAD - E
    src = jnp.concatenate([edge[0], jnp.zeros((pad,), jnp.int32)])
    dst = jnp.concatenate([edge[1], jnp.full((pad,), DUMP, jnp.int32)])
    return src.reshape(NS, CH, K), dst.reshape(NS, CH, K)


@jax.jit
def kernel(feat_A, feat_B, edge_e1, edge_e2, W_e1, b_e1, W_e2, b_e2):
    src1, dst1 = _pad_edges(edge_e1)
    src2, dst2 = _pad_edges(edge_e2)
    s1, d1, s2, d2 = _sc_aggregate(feat_A, feat_B, src1, dst1, src2, dst2)
    return _combine(s1, d1, s2, d2, W_e1, b_e1, W_e2, b_e2)
